# baseline (speedup 1.0000x reference)
import jax
import jax.numpy as jnp
from jax import lax
from jax.experimental import pallas as pl
from jax.experimental.pallas import tpu as pltpu
from jax.experimental.pallas import tpu_sc as plsc

_INV_SQRT2 = 0.7071067811865476

_NUM_CORES = 2
_NUM_SUBCORES = 16
_LANES = 16


def _tc_probs_body(x_ref, w1_ref, b1_ref, w2_ref, b2_ref, probs_ref):
    x = x_ref[...]
    h = jnp.dot(x, w1_ref[...], preferred_element_type=jnp.float32)
    h = h + b1_ref[...]
    h = 0.5 * h * (1.0 + jax.lax.erf(h * _INV_SQRT2))
    logits = jnp.dot(h, w2_ref[...], preferred_element_type=jnp.float32)
    logits = logits + b2_ref[...]
    m = jnp.max(logits, axis=-1, keepdims=True)
    e = jnp.exp(logits - m)
    s = jnp.sum(e, axis=-1, keepdims=True)
    probs_ref[...] = e / s


def _tc_probs(pooled_feat, W1, b1, W2, b2):
    B, D = pooled_feat.shape
    H = W1.shape[1]
    NE = W2.shape[1]
    BS = 2048
    grid = (B // BS,)
    return pl.pallas_call(
        _tc_probs_body,
        grid=grid,
        in_specs=[
            pl.BlockSpec((BS, D), lambda i: (i, 0)),
            pl.BlockSpec((D, H), lambda i: (0, 0)),
            pl.BlockSpec((1, H), lambda i: (0, 0)),
            pl.BlockSpec((H, NE), lambda i: (0, 0)),
            pl.BlockSpec((1, NE), lambda i: (0, 0)),
        ],
        out_specs=pl.BlockSpec((BS, NE), lambda i: (i, 0)),
        out_shape=jax.ShapeDtypeStruct((B, NE), jnp.float32),
        compiler_params=pltpu.CompilerParams(
            dimension_semantics=("parallel",),
        ),
    )(pooled_feat, W1, b1.reshape(1, H), W2, b2.reshape(1, NE))


def _make_sc_topk(B, NE):
    b_per_w = B // (_NUM_CORES * _NUM_SUBCORES)
    n_groups = b_per_w // _LANES

    def _sc_body(probs_hbm, rw_hbm, idx_hbm, probs_v, rw_v, idx_v):
        wid = lax.axis_index("s") * _NUM_CORES + lax.axis_index("c")
        base = wid * b_per_w
        pltpu.sync_copy(probs_hbm.at[pl.ds(base * NE, b_per_w * NE)], probs_v)

        lane = lax.iota(jnp.int32, _LANES)
        zeros16 = jnp.zeros((_LANES,), jnp.int32)
        ones16 = jnp.ones((_LANES,), jnp.int32)

        def group_step(g, carry):
            rows = g * _LANES + lane
            rows_ne = rows * NE
            rows2 = rows * 2

            def expert_step(e, tk):
                m1, i1, m2, i2 = tk
                ev = jnp.full((_LANES,), e, jnp.int32)
                v = plsc.load_gather(probs_v, [rows_ne + ev])
                c1 = v > m1
                c2 = jnp.logical_and(jnp.logical_not(c1), v > m2)
                m2n = jnp.where(c1, m1, jnp.where(c2, v, m2))
                i2n = jnp.where(c1, i1, jnp.where(c2, ev, i2))
                m1n = jnp.where(c1, v, m1)
                i1n = jnp.where(c1, ev, i1)
                return (m1n, i1n, m2n, i2n)

            neg = jnp.full((_LANES,), -1.0, jnp.float32)
            m1, i1, m2, i2 = lax.fori_loop(
                0, NE, expert_step, (neg, zeros16, neg, zeros16))
            denom = jnp.maximum(m1 + m2, 1e-6)
            plsc.store_scatter(rw_v, [rows2], m1 / denom)
            plsc.store_scatter(rw_v, [rows2 + ones16], m2 / denom)
            plsc.store_scatter(idx_v, [rows2], i1)
            plsc.store_scatter(idx_v, [rows2 + ones16], i2)
            return carry

        lax.fori_loop(0, n_groups, group_step, 0)
        pltpu.sync_copy(rw_v, rw_hbm.at[pl.ds(base * 2, b_per_w * 2)])
        pltpu.sync_copy(idx_v, idx_hbm.at[pl.ds(base * 2, b_per_w * 2)])

    return pl.kernel(
        _sc_body,
        out_type=[
            jax.ShapeDtypeStruct((B * 2,), jnp.float32),
            jax.ShapeDtypeStruct((B * 2,), jnp.int32),
        ],
        mesh=plsc.VectorSubcoreMesh(
            core_axis_name="c", subcore_axis_name="s",
            num_cores=_NUM_CORES, num_subcores=_NUM_SUBCORES),
        scratch_types=[
            pltpu.VMEM((b_per_w * NE,), jnp.float32),
            pltpu.VMEM((b_per_w * 2,), jnp.float32),
            pltpu.VMEM((b_per_w * 2,), jnp.int32),
        ],
        compiler_params=pltpu.CompilerParams(needs_layout_passes=False),
    )


def kernel(pooled_feat, W1, b1, W2, b2):
    B = pooled_feat.shape[0]
    NE = W2.shape[1]
    probs = _tc_probs(pooled_feat, W1, b1, W2, b2)
    rw_flat, idx_flat = _make_sc_topk(B, NE)(probs.reshape(B * NE))
    return (rw_flat.reshape(B, 2), idx_flat.reshape(B, 2), probs)


# speedup vs baseline: 1.0055x; 1.0055x over previous
import jax
import jax.numpy as jnp
from jax import lax
from jax.experimental import pallas as pl
from jax.experimental.pallas import tpu as pltpu
from jax.experimental.pallas import tpu_sc as plsc

_INV_SQRT2 = 0.7071067811865476

_NUM_CORES = 2
_NUM_SUBCORES = 16
_LANES = 16


def _tc_probs_body(x_ref, w1_ref, b1_ref, w2_ref, b2_ref, probs_ref):
    x = x_ref[...]
    h = jnp.dot(x, w1_ref[...], preferred_element_type=jnp.float32)
    h = h + b1_ref[...]
    h = 0.5 * h * (1.0 + jax.lax.erf(h * _INV_SQRT2))
    logits = jnp.dot(h, w2_ref[...], preferred_element_type=jnp.float32)
    logits = logits + b2_ref[...]
    m = jnp.max(logits, axis=-1, keepdims=True)
    e = jnp.exp(logits - m)
    s = jnp.sum(e, axis=-1, keepdims=True)
    probs_ref[...] = e / s


def _tc_probs(pooled_feat, W1, b1, W2, b2):
    B, D = pooled_feat.shape
    H = W1.shape[1]
    NE = W2.shape[1]
    BS = 2048
    grid = (B // BS,)
    return pl.pallas_call(
        _tc_probs_body,
        grid=grid,
        in_specs=[
            pl.BlockSpec((BS, D), lambda i: (i, 0)),
            pl.BlockSpec((D, H), lambda i: (0, 0)),
            pl.BlockSpec((1, H), lambda i: (0, 0)),
            pl.BlockSpec((H, NE), lambda i: (0, 0)),
            pl.BlockSpec((1, NE), lambda i: (0, 0)),
        ],
        out_specs=pl.BlockSpec((BS, NE), lambda i: (i, 0)),
        out_shape=jax.ShapeDtypeStruct((B, NE), jnp.float32),
        compiler_params=pltpu.CompilerParams(
            dimension_semantics=("parallel",),
        ),
    )(pooled_feat, W1, b1.reshape(1, H), W2, b2.reshape(1, NE))


def _make_sc_topk(B, NE):
    b_per_w = B // (_NUM_CORES * _NUM_SUBCORES)
    n_groups = b_per_w // _LANES

    def _sc_body(probs_hbm, rw_hbm, idx_hbm, probs_v, rw_v, idx_v):
        wid = lax.axis_index("s") * _NUM_CORES + lax.axis_index("c")
        base = wid * b_per_w
        pltpu.sync_copy(probs_hbm.at[pl.ds(base * NE, b_per_w * NE)], probs_v)

        lane = lax.iota(jnp.int32, _LANES)
        zeros16 = jnp.zeros((_LANES,), jnp.int32)
        ones16 = jnp.ones((_LANES,), jnp.int32)

        @plsc.parallel_loop(0, n_groups)
        def group_step(g):
            rows = g * _LANES + lane
            rows_ne = rows * NE
            rows2 = rows * 2

            m1 = plsc.load_gather(probs_v, [rows_ne])
            i1 = zeros16
            m2 = jnp.full((_LANES,), -1.0, jnp.float32)
            i2 = zeros16
            for e in range(1, NE):
                ev = jnp.full((_LANES,), e, jnp.int32)
                v = plsc.load_gather(probs_v, [rows_ne + ev])
                c1 = v > m1
                c2 = jnp.logical_and(jnp.logical_not(c1), v > m2)
                m2 = jnp.where(c1, m1, jnp.where(c2, v, m2))
                i2 = jnp.where(c1, i1, jnp.where(c2, ev, i2))
                m1 = jnp.where(c1, v, m1)
                i1 = jnp.where(c1, ev, i1)
            denom = jnp.maximum(m1 + m2, 1e-6)
            plsc.store_scatter(rw_v, [rows2], m1 / denom)
            plsc.store_scatter(rw_v, [rows2 + ones16], m2 / denom)
            plsc.store_scatter(idx_v, [rows2], i1)
            plsc.store_scatter(idx_v, [rows2 + ones16], i2)
        pltpu.sync_copy(rw_v, rw_hbm.at[pl.ds(base * 2, b_per_w * 2)])
        pltpu.sync_copy(idx_v, idx_hbm.at[pl.ds(base * 2, b_per_w * 2)])

    return pl.kernel(
        _sc_body,
        out_type=[
            jax.ShapeDtypeStruct((B * 2,), jnp.float32),
            jax.ShapeDtypeStruct((B * 2,), jnp.int32),
        ],
        mesh=plsc.VectorSubcoreMesh(
            core_axis_name="c", subcore_axis_name="s",
            num_cores=_NUM_CORES, num_subcores=_NUM_SUBCORES),
        scratch_types=[
            pltpu.VMEM((b_per_w * NE,), jnp.float32),
            pltpu.VMEM((b_per_w * 2,), jnp.float32),
            pltpu.VMEM((b_per_w * 2,), jnp.int32),
        ],
        compiler_params=pltpu.CompilerParams(needs_layout_passes=False),
    )


def kernel(pooled_feat, W1, b1, W2, b2):
    B = pooled_feat.shape[0]
    NE = W2.shape[1]
    probs = _tc_probs(pooled_feat, W1, b1, W2, b2)
    rw_flat, idx_flat = _make_sc_topk(B, NE)(probs.reshape(B * NE))
    return (rw_flat.reshape(B, 2), idx_flat.reshape(B, 2), probs)


# speedup vs baseline: 1.0983x; 1.0922x over previous
import jax
import jax.numpy as jnp
from jax import lax
from jax.experimental import pallas as pl
from jax.experimental.pallas import tpu as pltpu
from jax.experimental.pallas import tpu_sc as plsc

_INV_SQRT2 = 0.7071067811865476

_NUM_CORES = 2
_NUM_SUBCORES = 16
_LANES = 16


def _tc_probs_body(x_ref, w1_ref, b1_ref, w2_ref, b2_ref, probs_ref):
    x = x_ref[...]
    h = jnp.dot(x, w1_ref[...], preferred_element_type=jnp.float32)
    h = h + b1_ref[...]
    h = 0.5 * h * (1.0 + jax.lax.erf(h * _INV_SQRT2))
    logits = jnp.dot(h, w2_ref[...], preferred_element_type=jnp.float32)
    logits = logits + b2_ref[...]
    m = jnp.max(logits, axis=-1, keepdims=True)
    e = jnp.exp(logits - m)
    s = jnp.sum(e, axis=-1, keepdims=True)
    probs_ref[...] = e / s


def _tc_probs(pooled_feat, W1, b1, W2, b2):
    B, D = pooled_feat.shape
    H = W1.shape[1]
    NE = W2.shape[1]
    BS = 2048
    grid = (B // BS,)
    return pl.pallas_call(
        _tc_probs_body,
        grid=grid,
        in_specs=[
            pl.BlockSpec((BS, D), lambda i: (i, 0)),
            pl.BlockSpec((D, H), lambda i: (0, 0)),
            pl.BlockSpec((1, H), lambda i: (0, 0)),
            pl.BlockSpec((H, NE), lambda i: (0, 0)),
            pl.BlockSpec((1, NE), lambda i: (0, 0)),
        ],
        out_specs=pl.BlockSpec((BS, NE), lambda i: (i, 0)),
        out_shape=jax.ShapeDtypeStruct((B, NE), jnp.float32),
        compiler_params=pltpu.CompilerParams(
            dimension_semantics=("parallel",),
        ),
    )(pooled_feat, W1, b1.reshape(1, H), W2, b2.reshape(1, NE))


def _make_sc_topk(B, NE):
    b_per_w = B // (_NUM_CORES * _NUM_SUBCORES)
    n_groups = b_per_w // _LANES

    def _sc_body(probs_hbm, rw_hbm, idx_hbm, probs_v, rw_v, idx_v):
        wid = lax.axis_index("s") * _NUM_CORES + lax.axis_index("c")
        base = wid * b_per_w
        pltpu.sync_copy(probs_hbm.at[pl.ds(base * NE, b_per_w * NE)], probs_v)

        lane = lax.iota(jnp.int32, _LANES)
        zeros16 = jnp.zeros((_LANES,), jnp.int32)
        ones16 = jnp.ones((_LANES,), jnp.int32)

        @plsc.parallel_loop(0, n_groups, unroll=4)
        def group_step(g):
            rows = g * _LANES + lane
            rows_ne = rows * NE
            rows2 = rows * 2

            imask = jnp.full((_LANES,), ~0x3F, jnp.int32)
            v0 = plsc.load_gather(probs_v, [rows_ne])
            k1 = jnp.bitwise_or(
                jnp.bitwise_and(plsc.bitcast(v0, jnp.int32), imask),
                jnp.full((_LANES,), 63, jnp.int32))
            k2 = zeros16
            for e in range(1, NE):
                ev = jnp.full((_LANES,), e, jnp.int32)
                v = plsc.load_gather(probs_v, [rows_ne + ev])
                key = jnp.bitwise_or(
                    jnp.bitwise_and(plsc.bitcast(v, jnp.int32), imask),
                    jnp.full((_LANES,), 63 - e, jnp.int32))
                k2 = jnp.maximum(k2, jnp.minimum(k1, key))
                k1 = jnp.maximum(k1, key)
            c63 = jnp.full((_LANES,), 63, jnp.int32)
            i1 = c63 - jnp.bitwise_and(k1, c63)
            i2 = c63 - jnp.bitwise_and(k2, c63)
            p1 = plsc.load_gather(probs_v, [rows_ne + i1])
            p2 = plsc.load_gather(probs_v, [rows_ne + i2])
            denom = jnp.maximum(p1 + p2, 1e-6)
            plsc.store_scatter(rw_v, [rows2], p1 / denom)
            plsc.store_scatter(rw_v, [rows2 + ones16], p2 / denom)
            plsc.store_scatter(idx_v, [rows2], i1)
            plsc.store_scatter(idx_v, [rows2 + ones16], i2)
        pltpu.sync_copy(rw_v, rw_hbm.at[pl.ds(base * 2, b_per_w * 2)])
        pltpu.sync_copy(idx_v, idx_hbm.at[pl.ds(base * 2, b_per_w * 2)])

    return pl.kernel(
        _sc_body,
        out_type=[
            jax.ShapeDtypeStruct((B * 2,), jnp.float32),
            jax.ShapeDtypeStruct((B * 2,), jnp.int32),
        ],
        mesh=plsc.VectorSubcoreMesh(
            core_axis_name="c", subcore_axis_name="s",
            num_cores=_NUM_CORES, num_subcores=_NUM_SUBCORES),
        scratch_types=[
            pltpu.VMEM((b_per_w * NE,), jnp.float32),
            pltpu.VMEM((b_per_w * 2,), jnp.float32),
            pltpu.VMEM((b_per_w * 2,), jnp.int32),
        ],
        compiler_params=pltpu.CompilerParams(needs_layout_passes=False),
    )


def kernel(pooled_feat, W1, b1, W2, b2):
    B = pooled_feat.shape[0]
    NE = W2.shape[1]
    probs = _tc_probs(pooled_feat, W1, b1, W2, b2)
    rw_flat, idx_flat = _make_sc_topk(B, NE)(probs.reshape(B * NE))
    return (rw_flat.reshape(B, 2), idx_flat.reshape(B, 2), probs)


# speedup vs baseline: 1.8643x; 1.6975x over previous
import jax
import jax.numpy as jnp
from jax.experimental import pallas as pl
from jax.experimental.pallas import tpu as pltpu

_INV_SQRT2 = 0.7071067811865476


def _router_body(x_ref, w1_ref, b1_ref, w2_ref, b2_ref,
                 probs_ref, rw_ref, idx_ref):
    x = x_ref[...]
    h = jnp.dot(x, w1_ref[...], preferred_element_type=jnp.float32)
    h = h + b1_ref[...]
    h = 0.5 * h * (1.0 + jax.lax.erf(h * _INV_SQRT2))
    logits = jnp.dot(h, w2_ref[...], preferred_element_type=jnp.float32)
    logits = logits + b2_ref[...]

    m = jnp.max(logits, axis=-1, keepdims=True)
    e = jnp.exp(logits - m)
    s = jnp.sum(e, axis=-1, keepdims=True)
    probs = e / s
    probs_ref[...] = probs

    ne = probs.shape[-1]
    iota = jax.lax.broadcasted_iota(jnp.int32, probs.shape, 1)
    keys = ((probs.view(jnp.int32) & ~0x3F) | ((ne - 1) - iota))
    k1 = jnp.max(keys, axis=-1, keepdims=True)
    keys2 = jnp.where(keys == k1, 0, keys)
    k2 = jnp.max(keys2, axis=-1, keepdims=True)
    i1 = (ne - 1) - (k1 & 0x3F)
    i2 = (ne - 1) - (k2 & 0x3F)
    p1 = (k1 & ~0x3F).view(jnp.float32)
    p2 = (k2 & ~0x3F).view(jnp.float32)
    denom = jnp.maximum(p1 + p2, 1e-6)
    rw_ref[...] = jnp.concatenate([p1 / denom, p2 / denom], axis=-1)
    idx_ref[...] = jnp.concatenate([i1, i2], axis=-1)


def kernel(pooled_feat, W1, b1, W2, b2):
    B, D = pooled_feat.shape
    H = W1.shape[1]
    NE = W2.shape[1]
    BS = 2048
    grid = (B // BS,)

    probs, rw, idx = pl.pallas_call(
        _router_body,
        grid=grid,
        in_specs=[
            pl.BlockSpec((BS, D), lambda i: (i, 0)),
            pl.BlockSpec((D, H), lambda i: (0, 0)),
            pl.BlockSpec((1, H), lambda i: (0, 0)),
            pl.BlockSpec((H, NE), lambda i: (0, 0)),
            pl.BlockSpec((1, NE), lambda i: (0, 0)),
        ],
        out_specs=[
            pl.BlockSpec((BS, NE), lambda i: (i, 0)),
            pl.BlockSpec((BS, 2), lambda i: (i, 0)),
            pl.BlockSpec((BS, 2), lambda i: (i, 0)),
        ],
        out_shape=[
            jax.ShapeDtypeStruct((B, NE), jnp.float32),
            jax.ShapeDtypeStruct((B, 2), jnp.float32),
            jax.ShapeDtypeStruct((B, 2), jnp.int32),
        ],
        compiler_params=pltpu.CompilerParams(
            dimension_semantics=("parallel",),
        ),
    )(pooled_feat, W1, b1.reshape(1, H), W2, b2.reshape(1, NE))

    return (rw, idx, probs)


# speedup vs baseline: 1.9773x; 1.0606x over previous
import jax
import jax.numpy as jnp
from jax.experimental import pallas as pl
from jax.experimental.pallas import tpu as pltpu

_INV_SQRT2 = 0.7071067811865476


def _router_body(x_ref, w1_ref, b1_ref, w2_ref, b2_ref,
                 probs_ref, rw_ref, idx_ref):
    x = x_ref[...]
    h = jnp.dot(x, w1_ref[...], preferred_element_type=jnp.float32)
    h = h + b1_ref[...]
    h = 0.5 * h * (1.0 + jax.lax.erf(h * _INV_SQRT2))
    logits = jnp.dot(h, w2_ref[...], preferred_element_type=jnp.float32)
    logits = logits + b2_ref[...]

    m = jnp.max(logits, axis=-1, keepdims=True)
    e = jnp.exp(logits - m)
    s = jnp.sum(e, axis=-1, keepdims=True)
    probs_ref[...] = e / s

    ne = e.shape[-1]
    iota = jax.lax.broadcasted_iota(jnp.int32, e.shape, 1)
    keys = ((e.view(jnp.int32) & ~0x3F) | ((ne - 1) - iota)).view(jnp.float32)
    k1 = jnp.max(keys, axis=-1, keepdims=True)
    keys2 = jnp.where(keys == k1, 0.0, keys)
    k2 = jnp.max(keys2, axis=-1, keepdims=True)
    k1i = k1.view(jnp.int32)
    k2i = k2.view(jnp.int32)
    i1 = (ne - 1) - (k1i & 0x3F)
    i2 = (ne - 1) - (k2i & 0x3F)
    p1 = (k1i & ~0x3F).view(jnp.float32)
    p2 = (k2i & ~0x3F).view(jnp.float32)
    denom = jnp.maximum(p1 + p2, 1e-6)
    rw_ref[...] = jnp.concatenate([p1 / denom, p2 / denom], axis=-1)
    idx_ref[...] = jnp.concatenate([i1, i2], axis=-1)


def kernel(pooled_feat, W1, b1, W2, b2):
    B, D = pooled_feat.shape
    H = W1.shape[1]
    NE = W2.shape[1]
    BS = 2048
    grid = (B // BS,)

    probs, rw, idx = pl.pallas_call(
        _router_body,
        grid=grid,
        in_specs=[
            pl.BlockSpec((BS, D), lambda i: (i, 0)),
            pl.BlockSpec((D, H), lambda i: (0, 0)),
            pl.BlockSpec((1, H), lambda i: (0, 0)),
            pl.BlockSpec((H, NE), lambda i: (0, 0)),
            pl.BlockSpec((1, NE), lambda i: (0, 0)),
        ],
        out_specs=[
            pl.BlockSpec((BS, NE), lambda i: (i, 0)),
            pl.BlockSpec((BS, 2), lambda i: (i, 0)),
            pl.BlockSpec((BS, 2), lambda i: (i, 0)),
        ],
        out_shape=[
            jax.ShapeDtypeStruct((B, NE), jnp.float32),
            jax.ShapeDtypeStruct((B, 2), jnp.float32),
            jax.ShapeDtypeStruct((B, 2), jnp.int32),
        ],
        compiler_params=pltpu.CompilerParams(
            dimension_semantics=("parallel",),
        ),
    )(pooled_feat, W1, b1.reshape(1, H), W2, b2.reshape(1, NE))

    return (rw, idx, probs)
